# baseline (speedup 1.0000x reference)
import jax
import jax.numpy as jnp
from jax.experimental import pallas as pl
from jax.experimental.pallas import tpu as pltpu

_K = 1024
_C = 32
_BETA = 0.25
_TOKENS_PER_TILE = 512


def _vq_tile_kernel(x_ref, a_ref, cb_ref, e2_ref, cbt_ref, qz_ref, loss_ref):
    x = x_ref[0]
    cb = cb_ref[...]

    s = jax.lax.dot_general(
        cb, x, (((1,), (0,)), ((), ())),
        precision=jax.lax.Precision.DEFAULT,
        preferred_element_type=jnp.float32)
    a = a_ref[0]
    e2 = e2_ref[...]
    d = (a - 2.0 * s) + e2

    m = jnp.min(d, axis=0, keepdims=True)
    jidx = jax.lax.broadcasted_iota(jnp.int32, (_K, 1), 0).astype(jnp.float32)
    first = jnp.min(jnp.where(d == m, jidx, jnp.float32(_K)),
                    axis=0, keepdims=True)
    onehot = (jidx == first).astype(jnp.bfloat16)

    cbt = cbt_ref[...]
    cbt_hi = cbt.astype(jnp.bfloat16)
    cbt_lo = (cbt - cbt_hi.astype(jnp.float32)).astype(jnp.bfloat16)
    q_hi = jax.lax.dot_general(
        cbt_hi, onehot, (((1,), (0,)), ((), ())),
        precision=jax.lax.Precision.DEFAULT,
        preferred_element_type=jnp.float32)
    q_lo = jax.lax.dot_general(
        cbt_lo, onehot, (((1,), (0,)), ((), ())),
        precision=jax.lax.Precision.DEFAULT,
        preferred_element_type=jnp.float32)
    q = q_hi + q_lo

    diff = q - x
    qz_ref[0] = x + diff
    loss_ref[0, 0, 0, 0] = jnp.sum(diff * diff)


def kernel(inputs, codebook):
    B, C, T, H, W = inputs.shape
    thw = T * H * W
    x = inputs.reshape(B, C, thw)
    n_tiles = thw // _TOKENS_PER_TILE

    cbt = codebook.T

    a_tok = jnp.sum(
        jnp.transpose(inputs, (0, 2, 3, 4, 1)).reshape(-1, C) ** 2,
        axis=1, keepdims=True)
    a_b = a_tok.reshape(B, 1, thw)
    e2 = jnp.sum(codebook ** 2, axis=1).reshape(_K, 1)

    qz, loss_sum = pl.pallas_call(
        _vq_tile_kernel,
        grid=(B, n_tiles),
        in_specs=[
            pl.BlockSpec((1, C, _TOKENS_PER_TILE), lambda b, t: (b, 0, t)),
            pl.BlockSpec((1, 1, _TOKENS_PER_TILE), lambda b, t: (b, 0, t)),
            pl.BlockSpec((_K, _C), lambda b, t: (0, 0)),
            pl.BlockSpec((_K, 1), lambda b, t: (0, 0)),
            pl.BlockSpec((_C, _K), lambda b, t: (0, 0)),
        ],
        out_specs=[
            pl.BlockSpec((1, C, _TOKENS_PER_TILE), lambda b, t: (b, 0, t)),
            pl.BlockSpec((1, 1, 1, 1), lambda b, t: (b, t, 0, 0),
                         memory_space=pltpu.SMEM),
        ],
        out_shape=[
            jax.ShapeDtypeStruct((B, C, thw), jnp.float32),
            jax.ShapeDtypeStruct((B, n_tiles, 1, 1), jnp.float32),
        ],
        compiler_params=pltpu.CompilerParams(
            dimension_semantics=("parallel", "parallel")),
    )(x, a_b, codebook, e2, cbt)

    commitment_loss = jnp.sum(loss_sum) / jnp.float32(B * C * thw)
    vq_loss = commitment_loss * _BETA
    q_z = qz.reshape(B, C, T, H, W)
    perplexity = jnp.array([0.0], dtype=jnp.float32)
    return (q_z, vq_loss, commitment_loss, perplexity)


# speedup vs baseline: 1.1367x; 1.1367x over previous
import jax
import jax.numpy as jnp
from jax.experimental import pallas as pl
from jax.experimental.pallas import tpu as pltpu

_K = 1024
_C = 32
_BETA = 0.25
_TOKENS_PER_TILE = 1024


def _vq_tile_kernel(x_ref, a_ref, cb_ref, e2_ref, cbt_ref, qz_ref, loss_ref):
    x = x_ref[0]
    cb = cb_ref[...]

    s = jax.lax.dot_general(
        cb, x, (((1,), (0,)), ((), ())),
        precision=jax.lax.Precision.DEFAULT,
        preferred_element_type=jnp.float32)
    a = a_ref[0]
    e2 = e2_ref[...]
    d = (a - 2.0 * s) + e2

    m = jnp.min(d, axis=0, keepdims=True)
    jidx = jax.lax.broadcasted_iota(jnp.int32, (_K, 1), 0).astype(jnp.float32)
    first = jnp.min(jnp.where(d == m, jidx, jnp.float32(_K)),
                    axis=0, keepdims=True)
    onehot = (jidx == first).astype(jnp.bfloat16)

    cbt = cbt_ref[...]
    cbt_hi = cbt.astype(jnp.bfloat16)
    cbt_lo = (cbt - cbt_hi.astype(jnp.float32)).astype(jnp.bfloat16)
    q_hi = jax.lax.dot_general(
        cbt_hi, onehot, (((1,), (0,)), ((), ())),
        precision=jax.lax.Precision.DEFAULT,
        preferred_element_type=jnp.float32)
    q_lo = jax.lax.dot_general(
        cbt_lo, onehot, (((1,), (0,)), ((), ())),
        precision=jax.lax.Precision.DEFAULT,
        preferred_element_type=jnp.float32)
    q = q_hi + q_lo

    diff = q - x
    qz_ref[0] = x + diff
    loss_ref[0, 0, 0, 0] = jnp.sum(diff * diff)


def kernel(inputs, codebook):
    B, C, T, H, W = inputs.shape
    thw = T * H * W
    x = inputs.reshape(B, C, thw)
    n_tiles = thw // _TOKENS_PER_TILE

    cbt = codebook.T

    a_tok = jnp.sum(
        jnp.transpose(inputs, (0, 2, 3, 4, 1)).reshape(-1, C) ** 2,
        axis=1, keepdims=True)
    a_b = a_tok.reshape(B, 1, thw)
    e2 = jnp.sum(codebook ** 2, axis=1).reshape(_K, 1)

    qz, loss_sum = pl.pallas_call(
        _vq_tile_kernel,
        grid=(B, n_tiles),
        in_specs=[
            pl.BlockSpec((1, C, _TOKENS_PER_TILE), lambda b, t: (b, 0, t)),
            pl.BlockSpec((1, 1, _TOKENS_PER_TILE), lambda b, t: (b, 0, t)),
            pl.BlockSpec((_K, _C), lambda b, t: (0, 0)),
            pl.BlockSpec((_K, 1), lambda b, t: (0, 0)),
            pl.BlockSpec((_C, _K), lambda b, t: (0, 0)),
        ],
        out_specs=[
            pl.BlockSpec((1, C, _TOKENS_PER_TILE), lambda b, t: (b, 0, t)),
            pl.BlockSpec((1, 1, 1, 1), lambda b, t: (b, t, 0, 0),
                         memory_space=pltpu.SMEM),
        ],
        out_shape=[
            jax.ShapeDtypeStruct((B, C, thw), jnp.float32),
            jax.ShapeDtypeStruct((B, n_tiles, 1, 1), jnp.float32),
        ],
        compiler_params=pltpu.CompilerParams(
            dimension_semantics=("parallel", "parallel")),
    )(x, a_b, codebook, e2, cbt)

    commitment_loss = jnp.sum(loss_sum) / jnp.float32(B * C * thw)
    vq_loss = commitment_loss * _BETA
    q_z = qz.reshape(B, C, T, H, W)
    perplexity = jnp.array([0.0], dtype=jnp.float32)
    return (q_z, vq_loss, commitment_loss, perplexity)


# speedup vs baseline: 1.1682x; 1.0277x over previous
import jax
import jax.numpy as jnp
from jax.experimental import pallas as pl
from jax.experimental.pallas import tpu as pltpu

_K = 1024
_C = 32
_BETA = 0.25
_TOKENS_PER_TILE = 2048


def _vq_tile_kernel(x_ref, a_ref, cb_ref, e2_ref, cbt_ref, qz_ref, loss_ref):
    x = x_ref[0]
    cb = cb_ref[...]

    s = jax.lax.dot_general(
        cb, x, (((1,), (0,)), ((), ())),
        precision=jax.lax.Precision.DEFAULT,
        preferred_element_type=jnp.float32)
    a = a_ref[0]
    e2 = e2_ref[...]
    d = (a - 2.0 * s) + e2

    m = jnp.min(d, axis=0, keepdims=True)
    jidx = jax.lax.broadcasted_iota(jnp.int32, (_K, 1), 0).astype(jnp.float32)
    first = jnp.min(jnp.where(d == m, jidx, jnp.float32(_K)),
                    axis=0, keepdims=True)
    onehot = (jidx == first).astype(jnp.bfloat16)

    cbt = cbt_ref[...]
    cbt_hi = cbt.astype(jnp.bfloat16)
    cbt_lo = (cbt - cbt_hi.astype(jnp.float32)).astype(jnp.bfloat16)
    q_hi = jax.lax.dot_general(
        cbt_hi, onehot, (((1,), (0,)), ((), ())),
        precision=jax.lax.Precision.DEFAULT,
        preferred_element_type=jnp.float32)
    q_lo = jax.lax.dot_general(
        cbt_lo, onehot, (((1,), (0,)), ((), ())),
        precision=jax.lax.Precision.DEFAULT,
        preferred_element_type=jnp.float32)
    q = q_hi + q_lo

    diff = q - x
    qz_ref[0] = x + diff
    loss_ref[0, 0, 0, 0] = jnp.sum(diff * diff)


def kernel(inputs, codebook):
    B, C, T, H, W = inputs.shape
    thw = T * H * W
    x = inputs.reshape(B, C, thw)
    n_tiles = thw // _TOKENS_PER_TILE

    cbt = codebook.T

    a_tok = jnp.sum(
        jnp.transpose(inputs, (0, 2, 3, 4, 1)).reshape(-1, C) ** 2,
        axis=1, keepdims=True)
    a_b = a_tok.reshape(B, 1, thw)
    e2 = jnp.sum(codebook ** 2, axis=1).reshape(_K, 1)

    qz, loss_sum = pl.pallas_call(
        _vq_tile_kernel,
        grid=(B, n_tiles),
        in_specs=[
            pl.BlockSpec((1, C, _TOKENS_PER_TILE), lambda b, t: (b, 0, t)),
            pl.BlockSpec((1, 1, _TOKENS_PER_TILE), lambda b, t: (b, 0, t)),
            pl.BlockSpec((_K, _C), lambda b, t: (0, 0)),
            pl.BlockSpec((_K, 1), lambda b, t: (0, 0)),
            pl.BlockSpec((_C, _K), lambda b, t: (0, 0)),
        ],
        out_specs=[
            pl.BlockSpec((1, C, _TOKENS_PER_TILE), lambda b, t: (b, 0, t)),
            pl.BlockSpec((1, 1, 1, 1), lambda b, t: (b, t, 0, 0),
                         memory_space=pltpu.SMEM),
        ],
        out_shape=[
            jax.ShapeDtypeStruct((B, C, thw), jnp.float32),
            jax.ShapeDtypeStruct((B, n_tiles, 1, 1), jnp.float32),
        ],
        compiler_params=pltpu.CompilerParams(
            dimension_semantics=("parallel", "parallel")),
    )(x, a_b, codebook, e2, cbt)

    commitment_loss = jnp.sum(loss_sum) / jnp.float32(B * C * thw)
    vq_loss = commitment_loss * _BETA
    q_z = qz.reshape(B, C, T, H, W)
    perplexity = jnp.array([0.0], dtype=jnp.float32)
    return (q_z, vq_loss, commitment_loss, perplexity)


# speedup vs baseline: 1.1798x; 1.0099x over previous
import jax
import jax.numpy as jnp
from jax.experimental import pallas as pl
from jax.experimental.pallas import tpu as pltpu

_K = 1024
_C = 32
_BETA = 0.25
_TOKENS_PER_TILE = 4096


def _vq_tile_kernel(x_ref, a_ref, cb_ref, e2_ref, cbt_ref, qz_ref, loss_ref):
    x = x_ref[0]
    cb = cb_ref[...]

    s = jax.lax.dot_general(
        cb, x, (((1,), (0,)), ((), ())),
        precision=jax.lax.Precision.DEFAULT,
        preferred_element_type=jnp.float32)
    a = a_ref[0]
    e2 = e2_ref[...]
    d = (a - 2.0 * s) + e2

    m = jnp.min(d, axis=0, keepdims=True)
    jidx = jax.lax.broadcasted_iota(jnp.int32, (_K, 1), 0).astype(jnp.float32)
    first = jnp.min(jnp.where(d == m, jidx, jnp.float32(_K)),
                    axis=0, keepdims=True)
    onehot = (jidx == first).astype(jnp.bfloat16)

    cbt = cbt_ref[...]
    cbt_hi = cbt.astype(jnp.bfloat16)
    cbt_lo = (cbt - cbt_hi.astype(jnp.float32)).astype(jnp.bfloat16)
    q_hi = jax.lax.dot_general(
        cbt_hi, onehot, (((1,), (0,)), ((), ())),
        precision=jax.lax.Precision.DEFAULT,
        preferred_element_type=jnp.float32)
    q_lo = jax.lax.dot_general(
        cbt_lo, onehot, (((1,), (0,)), ((), ())),
        precision=jax.lax.Precision.DEFAULT,
        preferred_element_type=jnp.float32)
    q = q_hi + q_lo

    diff = q - x
    qz_ref[0] = x + diff
    loss_ref[0, 0, 0, 0] = jnp.sum(diff * diff)


def kernel(inputs, codebook):
    B, C, T, H, W = inputs.shape
    thw = T * H * W
    x = inputs.reshape(B, C, thw)
    n_tiles = thw // _TOKENS_PER_TILE

    cbt = codebook.T

    a_tok = jnp.sum(
        jnp.transpose(inputs, (0, 2, 3, 4, 1)).reshape(-1, C) ** 2,
        axis=1, keepdims=True)
    a_b = a_tok.reshape(B, 1, thw)
    e2 = jnp.sum(codebook ** 2, axis=1).reshape(_K, 1)

    qz, loss_sum = pl.pallas_call(
        _vq_tile_kernel,
        grid=(B, n_tiles),
        in_specs=[
            pl.BlockSpec((1, C, _TOKENS_PER_TILE), lambda b, t: (b, 0, t)),
            pl.BlockSpec((1, 1, _TOKENS_PER_TILE), lambda b, t: (b, 0, t)),
            pl.BlockSpec((_K, _C), lambda b, t: (0, 0)),
            pl.BlockSpec((_K, 1), lambda b, t: (0, 0)),
            pl.BlockSpec((_C, _K), lambda b, t: (0, 0)),
        ],
        out_specs=[
            pl.BlockSpec((1, C, _TOKENS_PER_TILE), lambda b, t: (b, 0, t)),
            pl.BlockSpec((1, 1, 1, 1), lambda b, t: (b, t, 0, 0),
                         memory_space=pltpu.SMEM),
        ],
        out_shape=[
            jax.ShapeDtypeStruct((B, C, thw), jnp.float32),
            jax.ShapeDtypeStruct((B, n_tiles, 1, 1), jnp.float32),
        ],
        compiler_params=pltpu.CompilerParams(
            dimension_semantics=("parallel", "parallel")),
    )(x, a_b, codebook, e2, cbt)

    commitment_loss = jnp.sum(loss_sum) / jnp.float32(B * C * thw)
    vq_loss = commitment_loss * _BETA
    q_z = qz.reshape(B, C, T, H, W)
    perplexity = jnp.array([0.0], dtype=jnp.float32)
    return (q_z, vq_loss, commitment_loss, perplexity)
